# baseline (speedup 1.0000x reference)
import functools

import jax
import jax.numpy as jnp
from jax import lax
from jax.experimental import pallas as pl
from jax.experimental.pallas import tpu as pltpu
from jax.experimental.pallas import tpu_sc as plsc

NC = 2
NS = 16
NW = NC * NS
MOLS = 256
EP = 163840
CG = 128
CWG = EP // NW // CG
CS = 64
CWS = EP // NW // CS
GRP = 4


def _proj_body(P, hp_ref, w_ref, out_ref):
    h = hp_ref[0]
    for p in range(1, P):
        h = h + hp_ref[p]
    out_ref[...] = jnp.dot(h, w_ref[...], preferred_element_type=jnp.float32)


def _node_proj(hparts, wcat, bn=2000):
    P, n, d = hparts.shape
    d2 = wcat.shape[1]
    return pl.pallas_call(
        functools.partial(_proj_body, P),
        grid=(n // bn,),
        in_specs=[
            pl.BlockSpec((P, bn, d), lambda i: (0, i, 0)),
            pl.BlockSpec((d, d2), lambda i: (0, 0)),
        ],
        out_specs=pl.BlockSpec((bn, d2), lambda i: (i, 0)),
        out_shape=jax.ShapeDtypeStruct((n, d2), jnp.float32),
    )(hparts, wcat)


def _sc_gather(table, ia2, ib2):
    n2, h = table.shape
    mesh = plsc.VectorSubcoreMesh(core_axis_name="c", subcore_axis_name="s")

    @functools.partial(
        pl.kernel,
        out_type=(jax.ShapeDtypeStruct((EP, h), jnp.float32),
                  jax.ShapeDtypeStruct((EP, h), jnp.float32)),
        mesh=mesh,
        compiler_params=pltpu.CompilerParams(use_tc_tiling_on_sc=False),
        scratch_types=[
            pltpu.VMEM((CWG, CG), jnp.int32),
            pltpu.VMEM((CWG, CG), jnp.int32),
            pltpu.VMEM((GRP, CG, h), jnp.float32),
            pltpu.VMEM((GRP, CG, h), jnp.float32),
            [pltpu.SemaphoreType.DMA] * GRP,
            [pltpu.SemaphoreType.DMA] * GRP,
            [pltpu.SemaphoreType.DMA] * GRP,
            [pltpu.SemaphoreType.DMA] * GRP,
        ],
    )
    def k(t_ref, ia_ref, ib_ref, oa_ref, ob_ref, iav, ibv, abuf, bbuf,
          ga, gb, sa, sb):
        ci = lax.axis_index("c")
        s = lax.axis_index("s")
        wid = s * NC + ci
        pltpu.sync_copy(ia_ref.at[pl.ds(wid * CWG, CWG)], iav)
        pltpu.sync_copy(ib_ref.at[pl.ds(wid * CWG, CWG)], ibv)
        base = wid * CWG

        def group(oj, carry):
            @pl.when(oj > 0)
            def _():
                for b in range(GRP):
                    pltpu.make_async_copy(
                        abuf.at[b], oa_ref.at[pl.ds(base * CG, CG)],
                        sa[b]).wait()
                    pltpu.make_async_copy(
                        bbuf.at[b], ob_ref.at[pl.ds(base * CG, CG)],
                        sb[b]).wait()
            descs = []
            for b in range(GRP):
                kk = GRP * oj + b
                descs.append(
                    (pltpu.async_copy(t_ref.at[iav.at[kk]], abuf.at[b], ga[b]),
                     pltpu.async_copy(t_ref.at[ibv.at[kk]], bbuf.at[b], gb[b])))
            for b in range(GRP):
                kk = GRP * oj + b
                da, db = descs[b]
                da.wait()
                db.wait()
                pltpu.async_copy(
                    abuf.at[b], oa_ref.at[pl.ds((base + kk) * CG, CG)], sa[b])
                pltpu.async_copy(
                    bbuf.at[b], ob_ref.at[pl.ds((base + kk) * CG, CG)], sb[b])
            return carry

        lax.fori_loop(0, CWG // GRP, group, 0)
        for b in range(GRP):
            pltpu.make_async_copy(
                abuf.at[b], oa_ref.at[pl.ds(base * CG, CG)], sa[b]).wait()
            pltpu.make_async_copy(
                bbuf.at[b], ob_ref.at[pl.ds(base * CG, CG)], sb[b]).wait()

    return k(table, ia2, ib2)


def _mlp_body(pa_ref, pb_ref, b1_ref, w2_ref, b2_ref, w3_ref, b3_ref, out_ref):
    m = jnp.maximum(pa_ref[...] + pb_ref[...] + b1_ref[...], 0.0)
    m = jnp.maximum(
        jnp.dot(m, w2_ref[...], preferred_element_type=jnp.float32) + b2_ref[...], 0.0)
    out_ref[...] = jnp.maximum(
        jnp.dot(m, w3_ref[...], preferred_element_type=jnp.float32) + b3_ref[...], 0.0)


def _edge_mlp(pre_a, pre_b, b1, w2, b2, w3, b3, be=4096):
    e, h = pre_a.shape
    d = w3.shape[1]
    return pl.pallas_call(
        _mlp_body,
        grid=(e // be,),
        in_specs=[
            pl.BlockSpec((be, h), lambda i: (i, 0)),
            pl.BlockSpec((be, h), lambda i: (i, 0)),
            pl.BlockSpec((1, h), lambda i: (0, 0)),
            pl.BlockSpec((h, h), lambda i: (0, 0)),
            pl.BlockSpec((1, h), lambda i: (0, 0)),
            pl.BlockSpec((h, d), lambda i: (0, 0)),
            pl.BlockSpec((1, d), lambda i: (0, 0)),
        ],
        out_specs=pl.BlockSpec((be, d), lambda i: (i, 0)),
        out_shape=jax.ShapeDtypeStruct((e, d), jnp.float32),
    )(pre_a, pre_b, b1, w2, b2, w3, b3)


def _sc_scatter(m3, dst2, zeros):
    d = m3.shape[1]
    na = zeros.shape[0]
    n = na - 8
    rw = (na // NS) // 8 * 8
    ztail = na - NS * rw
    wtail = n - NS * rw
    mesh = plsc.VectorSubcoreMesh(core_axis_name="c", subcore_axis_name="s")

    @functools.partial(
        pl.kernel,
        out_type=jax.ShapeDtypeStruct((NC, n, d), jnp.float32),
        mesh=mesh,
        scratch_types=[
            pltpu.VMEM((CWS, CS), jnp.int32),
            pltpu.VMEM((GRP, CS, d), jnp.float32),
            pltpu.VMEM_SHARED((na, d), jnp.float32),
            [pltpu.SemaphoreType.DMA] * GRP,
        ],
    )
    def k(m_ref, d_ref, z_ref, out_ref, idxv, rbuf, acc, g):
        ci = lax.axis_index("c")
        s = lax.axis_index("s")
        wid = s * NC + ci
        pltpu.sync_copy(z_ref.at[pl.ds(s * rw, rw)], acc.at[pl.ds(s * rw, rw)])
        if ztail:
            @pl.when(s == NS - 1)
            def _():
                pltpu.sync_copy(z_ref.at[pl.ds(NS * rw, ztail)],
                                acc.at[pl.ds(NS * rw, ztail)])
        pltpu.sync_copy(d_ref.at[pl.ds(wid * CWS, CWS)], idxv)
        plsc.subcore_barrier()
        base = wid * CWS

        def group(oj, carry):
            descs = []
            for b in range(GRP):
                kk = GRP * oj + b
                descs.append(pltpu.async_copy(
                    m_ref.at[pl.ds((base + kk) * CS, CS)], rbuf.at[b], g[b]))
            for b in range(GRP):
                kk = GRP * oj + b
                descs[b].wait()
                pltpu.sync_copy(rbuf.at[b], acc.at[idxv.at[kk]], add=True)
            return carry

        lax.fori_loop(0, CWS // GRP, group, 0)
        plsc.subcore_barrier()
        pltpu.sync_copy(acc.at[pl.ds(s * rw, rw)], out_ref.at[ci, pl.ds(s * rw, rw)])
        if wtail:
            @pl.when(s == NS - 1)
            def _():
                pltpu.sync_copy(acc.at[pl.ds(NS * rw, wtail)],
                                out_ref.at[ci, pl.ds(NS * rw, wtail)])

    return k(m3, dst2, zeros)


def _readout_body(hp_ref, mol_ref, w1_ref, b1_ref, w2_ref, b2_ref, w3_ref,
                  b3_ref, out_ref):
    @pl.when(pl.program_id(0) == 0)
    def _():
        out_ref[...] = jnp.zeros_like(out_ref)

    h = hp_ref[0] + hp_ref[1]
    r = jnp.maximum(
        jnp.dot(h, w1_ref[...], preferred_element_type=jnp.float32) + b1_ref[...], 0.0)
    r = jnp.maximum(
        jnp.dot(r, w2_ref[...], preferred_element_type=jnp.float32) + b2_ref[...], 0.0)
    r = jnp.maximum(
        jnp.dot(r, w3_ref[...], preferred_element_type=jnp.float32) + b3_ref[...], 0.0)
    mol = mol_ref[0, 0, :]
    rows = lax.broadcasted_iota(jnp.int32, (MOLS, mol.shape[0]), 0)
    onehot = (rows == mol[None, :]).astype(jnp.float32)
    out_ref[...] += jnp.dot(onehot, r, preferred_element_type=jnp.float32)


def _readout(hparts, mol3, w1, b1, w2, b2, w3, b3, bn=1000):
    P, n, d = hparts.shape
    h = w1.shape[1]
    o = w3.shape[1]
    g = n // bn
    return pl.pallas_call(
        _readout_body,
        grid=(g,),
        in_specs=[
            pl.BlockSpec((P, bn, d), lambda i: (0, i, 0)),
            pl.BlockSpec((1, 1, bn), lambda i: (i, 0, 0)),
            pl.BlockSpec((d, h), lambda i: (0, 0)),
            pl.BlockSpec((1, h), lambda i: (0, 0)),
            pl.BlockSpec((h, h), lambda i: (0, 0)),
            pl.BlockSpec((1, h), lambda i: (0, 0)),
            pl.BlockSpec((h, o), lambda i: (0, 0)),
            pl.BlockSpec((1, o), lambda i: (0, 0)),
        ],
        out_specs=pl.BlockSpec((MOLS, o), lambda i: (0, 0)),
        out_shape=jax.ShapeDtypeStruct((MOLS, o), jnp.float32),
    )(hparts, mol3, w1, b1, w2, b2, w3, b3)


def kernel(x, edge_index, mol_ids, msg_W1, msg_b1, msg_W2, msg_b2, msg_W3,
           msg_b3, fc1_W, fc1_b, fc2_W, fc2_b, out_W, out_b):
    n, d = x.shape
    e = edge_index.shape[1]
    steps, hid, _ = msg_W1.shape

    src = edge_index[0]
    dst = edge_index[1]
    pad = EP - e
    padz = jnp.zeros((pad,), jnp.int32)
    ia2 = jnp.concatenate([dst * 2, padz]).reshape(EP // CG, CG)
    ib2 = jnp.concatenate([src * 2 + 1, padz]).reshape(EP // CG, CG)
    dst2 = jnp.concatenate([dst, jnp.full((pad,), n, jnp.int32)]).reshape(EP // CS, CS)
    zeros = jnp.zeros((n + 8, d), jnp.float32)

    hparts = x[None]
    for s in range(steps):
        w1 = msg_W1[s]
        wcat = jnp.concatenate([w1[:, :d].T, w1[:, d:].T], axis=1)
        ab = _node_proj(hparts, wcat)
        table = ab.reshape(2 * n, hid)
        pre_a, pre_b = _sc_gather(table, ia2, ib2)
        m3 = _edge_mlp(pre_a, pre_b, msg_b1[s][None, :], msg_W2[s].T,
                       msg_b2[s][None, :], msg_W3[s].T, msg_b3[s][None, :])
        hparts = _sc_scatter(m3, dst2, zeros)

    mol3 = mol_ids.reshape(10, 1, n // 10)
    return _readout(hparts, mol3, fc1_W.T, fc1_b[None, :], fc2_W.T,
                    fc2_b[None, :], out_W.T, out_b[None, :])


# speedup vs baseline: 1.3054x; 1.3054x over previous
import functools

import jax
import jax.numpy as jnp
from jax import lax
from jax.experimental import pallas as pl
from jax.experimental.pallas import tpu as pltpu
from jax.experimental.pallas import tpu_sc as plsc

NC = 2
NS = 16
NW = NC * NS
MOLS = 256
EP = 163840
CG = 128
CWG = EP // NW // CG
CS = 64
CWS = EP // NW // CS
GRP = 4


def _proj_body(P, hp_ref, w_ref, out_ref):
    h = hp_ref[0]
    for p in range(1, P):
        h = h + hp_ref[p]
    out_ref[...] = jnp.dot(h, w_ref[...], preferred_element_type=jnp.float32)


def _node_proj(hparts, wcat, bn=2000):
    P, n, d = hparts.shape
    d2 = wcat.shape[1]
    return pl.pallas_call(
        functools.partial(_proj_body, P),
        grid=(n // bn,),
        in_specs=[
            pl.BlockSpec((P, bn, d), lambda i: (0, i, 0)),
            pl.BlockSpec((d, d2), lambda i: (0, 0)),
        ],
        out_specs=pl.BlockSpec((bn, d2), lambda i: (i, 0)),
        out_shape=jax.ShapeDtypeStruct((n, d2), jnp.float32),
    )(hparts, wcat)


def _sc_gather(table, ia2, ib2):
    n2, h = table.shape
    mesh = plsc.VectorSubcoreMesh(core_axis_name="c", subcore_axis_name="s")

    @functools.partial(
        pl.kernel,
        out_type=(jax.ShapeDtypeStruct((EP, h), jnp.float32),
                  jax.ShapeDtypeStruct((EP, h), jnp.float32)),
        mesh=mesh,
        compiler_params=pltpu.CompilerParams(use_tc_tiling_on_sc=False),
        scratch_types=[
            pltpu.VMEM((CWG, CG), jnp.int32),
            pltpu.VMEM((CWG, CG), jnp.int32),
            pltpu.VMEM((GRP, CG, h), jnp.float32),
            pltpu.VMEM((GRP, CG, h), jnp.float32),
            [pltpu.SemaphoreType.DMA] * GRP,
            [pltpu.SemaphoreType.DMA] * GRP,
            [pltpu.SemaphoreType.DMA] * GRP,
            [pltpu.SemaphoreType.DMA] * GRP,
        ],
    )
    def k(t_ref, ia_ref, ib_ref, oa_ref, ob_ref, iav, ibv, abuf, bbuf,
          ga, gb, sa, sb):
        ci = lax.axis_index("c")
        s = lax.axis_index("s")
        wid = s * NC + ci
        pltpu.sync_copy(ia_ref.at[pl.ds(wid * CWG, CWG)], iav)
        pltpu.sync_copy(ib_ref.at[pl.ds(wid * CWG, CWG)], ibv)
        base = wid * CWG

        def group(oj, carry):
            @pl.when(oj > 0)
            def _():
                for b in range(GRP):
                    pltpu.make_async_copy(
                        abuf.at[b], oa_ref.at[pl.ds(base * CG, CG)],
                        sa[b]).wait()
                    pltpu.make_async_copy(
                        bbuf.at[b], ob_ref.at[pl.ds(base * CG, CG)],
                        sb[b]).wait()
            descs = []
            for b in range(GRP):
                kk = GRP * oj + b
                descs.append(
                    (pltpu.async_copy(t_ref.at[iav.at[kk]], abuf.at[b], ga[b]),
                     pltpu.async_copy(t_ref.at[ibv.at[kk]], bbuf.at[b], gb[b])))
            for b in range(GRP):
                kk = GRP * oj + b
                da, db = descs[b]
                da.wait()
                db.wait()
                pltpu.async_copy(
                    abuf.at[b], oa_ref.at[pl.ds((base + kk) * CG, CG)], sa[b])
                pltpu.async_copy(
                    bbuf.at[b], ob_ref.at[pl.ds((base + kk) * CG, CG)], sb[b])
            return carry

        lax.fori_loop(0, CWG // GRP, group, 0)
        for b in range(GRP):
            pltpu.make_async_copy(
                abuf.at[b], oa_ref.at[pl.ds(base * CG, CG)], sa[b]).wait()
            pltpu.make_async_copy(
                bbuf.at[b], ob_ref.at[pl.ds(base * CG, CG)], sb[b]).wait()

    return k(table, ia2, ib2)


def _mlp_body(h, pa_ref, pb_ref, b1_ref, w2_ref, b2_ref, w3_ref, b3_ref,
              oa_ref, ob_ref):
    z = jnp.maximum(pa_ref[...] + pb_ref[...] + b1_ref[...], 0.0)
    m2 = jnp.maximum(
        jnp.dot(z, w2_ref[...], preferred_element_type=jnp.float32) + b2_ref[...], 0.0)
    me = m2[:, :h]
    mo = m2[:, h:]
    oa_ref[...] = jnp.maximum(
        jnp.dot(me, w3_ref[...], preferred_element_type=jnp.float32) + b3_ref[...], 0.0)
    ob_ref[...] = jnp.maximum(
        jnp.dot(mo, w3_ref[...], preferred_element_type=jnp.float32) + b3_ref[...], 0.0)


def _edge_mlp(pre_a, pre_b, b1c, w2bd, b2c, w3, b3, be=2048):
    ep2, h2 = pre_a.shape
    h = h2 // 2
    d = w3.shape[1]
    sds = jax.ShapeDtypeStruct((ep2, d), jnp.float32)
    return pl.pallas_call(
        functools.partial(_mlp_body, h),
        grid=(ep2 // be,),
        in_specs=[
            pl.BlockSpec((be, h2), lambda i: (i, 0)),
            pl.BlockSpec((be, h2), lambda i: (i, 0)),
            pl.BlockSpec((1, h2), lambda i: (0, 0)),
            pl.BlockSpec((h2, h2), lambda i: (0, 0)),
            pl.BlockSpec((1, h2), lambda i: (0, 0)),
            pl.BlockSpec((h, d), lambda i: (0, 0)),
            pl.BlockSpec((1, d), lambda i: (0, 0)),
        ],
        out_specs=[pl.BlockSpec((be, d), lambda i: (i, 0)),
                   pl.BlockSpec((be, d), lambda i: (i, 0))],
        out_shape=[sds, sds],
    )(pre_a, pre_b, b1c, w2bd, b2c, w3, b3)


def _sc_scatter(m3a, m3b, dste2, dsto2, zeros):
    d = m3a.shape[1]
    na = zeros.shape[0]
    n = na - 8
    cpw = m3a.shape[0] // NW // CS
    rw = (na // NS) // 8 * 8
    ztail = na - NS * rw
    wtail = n - NS * rw
    mesh = plsc.VectorSubcoreMesh(core_axis_name="c", subcore_axis_name="s")

    @functools.partial(
        pl.kernel,
        out_type=jax.ShapeDtypeStruct((NC, n, d), jnp.float32),
        mesh=mesh,
        scratch_types=[
            pltpu.VMEM((cpw, CS), jnp.int32),
            pltpu.VMEM((cpw, CS), jnp.int32),
            pltpu.VMEM((GRP, CS, d), jnp.float32),
            pltpu.VMEM_SHARED((na, d), jnp.float32),
            [pltpu.SemaphoreType.DMA] * GRP,
        ],
    )
    def k(ma_ref, mb_ref, de_ref, do_ref, z_ref, out_ref, idxe, idxo, rbuf,
          acc, g):
        ci = lax.axis_index("c")
        s = lax.axis_index("s")
        wid = s * NC + ci
        pltpu.sync_copy(z_ref.at[pl.ds(s * rw, rw)], acc.at[pl.ds(s * rw, rw)])
        if ztail:
            @pl.when(s == NS - 1)
            def _():
                pltpu.sync_copy(z_ref.at[pl.ds(NS * rw, ztail)],
                                acc.at[pl.ds(NS * rw, ztail)])
        pltpu.sync_copy(de_ref.at[pl.ds(wid * cpw, cpw)], idxe)
        pltpu.sync_copy(do_ref.at[pl.ds(wid * cpw, cpw)], idxo)
        plsc.subcore_barrier()
        base = wid * cpw

        for m_ref, idxv in ((ma_ref, idxe), (mb_ref, idxo)):
            def group(oj, carry):
                descs = []
                for b in range(GRP):
                    kk = GRP * oj + b
                    descs.append(pltpu.async_copy(
                        m_ref.at[pl.ds((base + kk) * CS, CS)], rbuf.at[b],
                        g[b]))
                for b in range(GRP):
                    kk = GRP * oj + b
                    descs[b].wait()
                    pltpu.sync_copy(rbuf.at[b], acc.at[idxv.at[kk]], add=True)
                return carry

            lax.fori_loop(0, cpw // GRP, group, 0)
        plsc.subcore_barrier()
        pltpu.sync_copy(acc.at[pl.ds(s * rw, rw)], out_ref.at[ci, pl.ds(s * rw, rw)])
        if wtail:
            @pl.when(s == NS - 1)
            def _():
                pltpu.sync_copy(acc.at[pl.ds(NS * rw, wtail)],
                                out_ref.at[ci, pl.ds(NS * rw, wtail)])

    return k(m3a, m3b, dste2, dsto2, zeros)


def _readout_body(hp_ref, mol_ref, w1_ref, b1_ref, w2_ref, b2_ref, w3_ref,
                  b3_ref, out_ref):
    @pl.when(pl.program_id(0) == 0)
    def _():
        out_ref[...] = jnp.zeros_like(out_ref)

    h = hp_ref[0] + hp_ref[1]
    r = jnp.maximum(
        jnp.dot(h, w1_ref[...], preferred_element_type=jnp.float32) + b1_ref[...], 0.0)
    r = jnp.maximum(
        jnp.dot(r, w2_ref[...], preferred_element_type=jnp.float32) + b2_ref[...], 0.0)
    r = jnp.maximum(
        jnp.dot(r, w3_ref[...], preferred_element_type=jnp.float32) + b3_ref[...], 0.0)
    mol = mol_ref[0, 0, :]
    rows = lax.broadcasted_iota(jnp.int32, (MOLS, mol.shape[0]), 0)
    onehot = (rows == mol[None, :]).astype(jnp.float32)
    out_ref[...] += jnp.dot(onehot, r, preferred_element_type=jnp.float32)


def _readout(hparts, mol3, w1, b1, w2, b2, w3, b3, bn=1000):
    P, n, d = hparts.shape
    h = w1.shape[1]
    o = w3.shape[1]
    g = n // bn
    return pl.pallas_call(
        _readout_body,
        grid=(g,),
        in_specs=[
            pl.BlockSpec((P, bn, d), lambda i: (0, i, 0)),
            pl.BlockSpec((1, 1, bn), lambda i: (i, 0, 0)),
            pl.BlockSpec((d, h), lambda i: (0, 0)),
            pl.BlockSpec((1, h), lambda i: (0, 0)),
            pl.BlockSpec((h, h), lambda i: (0, 0)),
            pl.BlockSpec((1, h), lambda i: (0, 0)),
            pl.BlockSpec((h, o), lambda i: (0, 0)),
            pl.BlockSpec((1, o), lambda i: (0, 0)),
        ],
        out_specs=pl.BlockSpec((MOLS, o), lambda i: (0, 0)),
        out_shape=jax.ShapeDtypeStruct((MOLS, o), jnp.float32),
    )(hparts, mol3, w1, b1, w2, b2, w3, b3)


def kernel(x, edge_index, mol_ids, msg_W1, msg_b1, msg_W2, msg_b2, msg_W3,
           msg_b3, fc1_W, fc1_b, fc2_W, fc2_b, out_W, out_b):
    n, d = x.shape
    e = edge_index.shape[1]
    steps, hid, _ = msg_W1.shape

    src = edge_index[0]
    dst = edge_index[1]
    pad = EP - e
    padz = jnp.zeros((pad,), jnp.int32)
    ia2 = jnp.concatenate([dst * 2, padz]).reshape(EP // CG, CG)
    ib2 = jnp.concatenate([src * 2 + 1, padz]).reshape(EP // CG, CG)
    dstp = jnp.concatenate([dst, jnp.full((pad,), n, jnp.int32)])
    dste2 = dstp[0::2].reshape(EP // 2 // CS, CS)
    dsto2 = dstp[1::2].reshape(EP // 2 // CS, CS)
    zeros = jnp.zeros((n + 8, d), jnp.float32)

    hparts = x[None]
    for s in range(steps):
        w1 = msg_W1[s]
        wcat = jnp.concatenate([w1[:, :d].T, w1[:, d:].T], axis=1)
        ab = _node_proj(hparts, wcat)
        table = ab.reshape(2 * n, hid)
        pre_a, pre_b = _sc_gather(table, ia2, ib2)
        pap = pre_a.reshape(EP // 2, 2 * hid)
        pbp = pre_b.reshape(EP // 2, 2 * hid)
        w2t = msg_W2[s].T
        w2bd = jnp.zeros((2 * hid, 2 * hid), jnp.float32)
        w2bd = w2bd.at[:hid, :hid].set(w2t).at[hid:, hid:].set(w2t)
        b1c = jnp.tile(msg_b1[s], 2)[None, :]
        b2c = jnp.tile(msg_b2[s], 2)[None, :]
        m3a, m3b = _edge_mlp(pap, pbp, b1c, w2bd, b2c, msg_W3[s].T,
                             msg_b3[s][None, :])
        hparts = _sc_scatter(m3a, m3b, dste2, dsto2, zeros)

    mol3 = mol_ids.reshape(10, 1, n // 10)
    return _readout(hparts, mol3, fc1_W.T, fc1_b[None, :], fc2_W.T,
                    fc2_b[None, :], out_W.T, out_b[None, :])


# speedup vs baseline: 2.5598x; 1.9609x over previous
import functools

import jax
import jax.numpy as jnp
from jax import lax
from jax.experimental import pallas as pl
from jax.experimental.pallas import tpu as pltpu
from jax.experimental.pallas import tpu_sc as plsc

NC = 2
NS = 16
NW = NC * NS
MOLS = 256
EP = 163840
CG = 64
CWG = EP // NW // CG
CS = 64
CWS = EP // NW // CS
GRP = 4


def _proj_body(P, hp_ref, w_ref, out_ref):
    h = hp_ref[0]
    for p in range(1, P):
        h = h + hp_ref[p]
    out_ref[...] = jnp.dot(h, w_ref[...], preferred_element_type=jnp.float32)


def _node_proj(hparts, wcat, bn=2000):
    P, n, d = hparts.shape
    d2 = wcat.shape[1]
    return pl.pallas_call(
        functools.partial(_proj_body, P),
        grid=(n // bn,),
        in_specs=[
            pl.BlockSpec((P, bn, d), lambda i: (0, i, 0)),
            pl.BlockSpec((d, d2), lambda i: (0, 0)),
        ],
        out_specs=pl.BlockSpec((bn, d2), lambda i: (i, 0)),
        out_shape=jax.ShapeDtypeStruct((n, d2), jnp.float32),
    )(hparts, wcat)


def _sc_gather(table, ia2, ib2):
    n2, h = table.shape
    trows = (n2 // NS) // 8 * 8
    ttail = n2 - NS * trows
    mesh = plsc.VectorSubcoreMesh(core_axis_name="c", subcore_axis_name="s")

    @functools.partial(
        pl.kernel,
        out_type=(jax.ShapeDtypeStruct((EP, h), jnp.float32),
                  jax.ShapeDtypeStruct((EP, h), jnp.float32)),
        mesh=mesh,
        compiler_params=pltpu.CompilerParams(use_tc_tiling_on_sc=False),
        scratch_types=[
            pltpu.VMEM((CWG, CG), jnp.int32),
            pltpu.VMEM((CWG, CG), jnp.int32),
            pltpu.VMEM((GRP, CG, h), jnp.float32),
            pltpu.VMEM((GRP, CG, h), jnp.float32),
            pltpu.VMEM_SHARED((n2, h), jnp.float32),
            [pltpu.SemaphoreType.DMA] * GRP,
            [pltpu.SemaphoreType.DMA] * GRP,
            [pltpu.SemaphoreType.DMA] * GRP,
            [pltpu.SemaphoreType.DMA] * GRP,
        ],
    )
    def k(t_ref, ia_ref, ib_ref, oa_ref, ob_ref, iav, ibv, abuf, bbuf, tsh,
          ga, gb, sa, sb):
        ci = lax.axis_index("c")
        s = lax.axis_index("s")
        wid = s * NC + ci
        pltpu.sync_copy(t_ref.at[pl.ds(s * trows, trows)],
                        tsh.at[pl.ds(s * trows, trows)])
        if ttail:
            @pl.when(s == NS - 1)
            def _():
                pltpu.sync_copy(t_ref.at[pl.ds(NS * trows, ttail)],
                                tsh.at[pl.ds(NS * trows, ttail)])
        pltpu.sync_copy(ia_ref.at[pl.ds(wid * CWG, CWG)], iav)
        pltpu.sync_copy(ib_ref.at[pl.ds(wid * CWG, CWG)], ibv)
        plsc.subcore_barrier()
        base = wid * CWG

        def group(oj, carry):
            @pl.when(oj > 0)
            def _():
                for b in range(GRP):
                    pltpu.make_async_copy(
                        abuf.at[b], oa_ref.at[pl.ds(base * CG, CG)],
                        sa[b]).wait()
                    pltpu.make_async_copy(
                        bbuf.at[b], ob_ref.at[pl.ds(base * CG, CG)],
                        sb[b]).wait()
            descs = []
            for b in range(GRP):
                kk = GRP * oj + b
                descs.append(
                    (pltpu.async_copy(tsh.at[iav.at[kk]], abuf.at[b], ga[b]),
                     pltpu.async_copy(tsh.at[ibv.at[kk]], bbuf.at[b], gb[b])))
            for b in range(GRP):
                kk = GRP * oj + b
                da, db = descs[b]
                da.wait()
                db.wait()
                pltpu.async_copy(
                    abuf.at[b], oa_ref.at[pl.ds((base + kk) * CG, CG)], sa[b])
                pltpu.async_copy(
                    bbuf.at[b], ob_ref.at[pl.ds((base + kk) * CG, CG)], sb[b])
            return carry

        lax.fori_loop(0, CWG // GRP, group, 0)
        for b in range(GRP):
            pltpu.make_async_copy(
                abuf.at[b], oa_ref.at[pl.ds(base * CG, CG)], sa[b]).wait()
            pltpu.make_async_copy(
                bbuf.at[b], ob_ref.at[pl.ds(base * CG, CG)], sb[b]).wait()

    return k(table, ia2, ib2)


def _mlp_body(h, pa_ref, pb_ref, b1_ref, w2_ref, b2_ref, w3_ref, b3_ref,
              oa_ref, ob_ref):
    z = jnp.maximum(pa_ref[...] + pb_ref[...] + b1_ref[...], 0.0)
    m2 = jnp.maximum(
        jnp.dot(z, w2_ref[...], preferred_element_type=jnp.float32) + b2_ref[...], 0.0)
    me = m2[:, :h]
    mo = m2[:, h:]
    oa_ref[...] = jnp.maximum(
        jnp.dot(me, w3_ref[...], preferred_element_type=jnp.float32) + b3_ref[...], 0.0)
    ob_ref[...] = jnp.maximum(
        jnp.dot(mo, w3_ref[...], preferred_element_type=jnp.float32) + b3_ref[...], 0.0)


def _edge_mlp(pre_a, pre_b, b1c, w2bd, b2c, w3, b3, be=2048):
    ep2, h2 = pre_a.shape
    h = h2 // 2
    d = w3.shape[1]
    sds = jax.ShapeDtypeStruct((ep2, d), jnp.float32)
    return pl.pallas_call(
        functools.partial(_mlp_body, h),
        grid=(ep2 // be,),
        in_specs=[
            pl.BlockSpec((be, h2), lambda i: (i, 0)),
            pl.BlockSpec((be, h2), lambda i: (i, 0)),
            pl.BlockSpec((1, h2), lambda i: (0, 0)),
            pl.BlockSpec((h2, h2), lambda i: (0, 0)),
            pl.BlockSpec((1, h2), lambda i: (0, 0)),
            pl.BlockSpec((h, d), lambda i: (0, 0)),
            pl.BlockSpec((1, d), lambda i: (0, 0)),
        ],
        out_specs=[pl.BlockSpec((be, d), lambda i: (i, 0)),
                   pl.BlockSpec((be, d), lambda i: (i, 0))],
        out_shape=[sds, sds],
    )(pre_a, pre_b, b1c, w2bd, b2c, w3, b3)


def _sc_scatter(m3a, m3b, dste2, dsto2, zeros):
    d = m3a.shape[1]
    na = zeros.shape[0]
    n = na - 8
    cpw = m3a.shape[0] // NW // CS
    rw = (na // NS) // 8 * 8
    ztail = na - NS * rw
    wtail = n - NS * rw
    mesh = plsc.VectorSubcoreMesh(core_axis_name="c", subcore_axis_name="s")

    @functools.partial(
        pl.kernel,
        out_type=jax.ShapeDtypeStruct((NC, n, d), jnp.float32),
        mesh=mesh,
        scratch_types=[
            pltpu.VMEM((cpw, CS), jnp.int32),
            pltpu.VMEM((cpw, CS), jnp.int32),
            pltpu.VMEM((GRP, CS, d), jnp.float32),
            pltpu.VMEM_SHARED((na, d), jnp.float32),
            [pltpu.SemaphoreType.DMA] * GRP,
        ],
    )
    def k(ma_ref, mb_ref, de_ref, do_ref, z_ref, out_ref, idxe, idxo, rbuf,
          acc, g):
        ci = lax.axis_index("c")
        s = lax.axis_index("s")
        wid = s * NC + ci
        pltpu.sync_copy(z_ref.at[pl.ds(s * rw, rw)], acc.at[pl.ds(s * rw, rw)])
        if ztail:
            @pl.when(s == NS - 1)
            def _():
                pltpu.sync_copy(z_ref.at[pl.ds(NS * rw, ztail)],
                                acc.at[pl.ds(NS * rw, ztail)])
        pltpu.sync_copy(de_ref.at[pl.ds(wid * cpw, cpw)], idxe)
        pltpu.sync_copy(do_ref.at[pl.ds(wid * cpw, cpw)], idxo)
        plsc.subcore_barrier()
        base = wid * cpw

        for m_ref, idxv in ((ma_ref, idxe), (mb_ref, idxo)):
            def group(oj, carry):
                descs = []
                for b in range(GRP):
                    kk = GRP * oj + b
                    descs.append(pltpu.async_copy(
                        m_ref.at[pl.ds((base + kk) * CS, CS)], rbuf.at[b],
                        g[b]))
                for b in range(GRP):
                    kk = GRP * oj + b
                    descs[b].wait()
                    pltpu.sync_copy(rbuf.at[b], acc.at[idxv.at[kk]], add=True)
                return carry

            lax.fori_loop(0, cpw // GRP, group, 0)
        plsc.subcore_barrier()
        pltpu.sync_copy(acc.at[pl.ds(s * rw, rw)], out_ref.at[ci, pl.ds(s * rw, rw)])
        if wtail:
            @pl.when(s == NS - 1)
            def _():
                pltpu.sync_copy(acc.at[pl.ds(NS * rw, wtail)],
                                out_ref.at[ci, pl.ds(NS * rw, wtail)])

    return k(m3a, m3b, dste2, dsto2, zeros)


def _readout_body(hp_ref, mol_ref, w1_ref, b1_ref, w2_ref, b2_ref, w3_ref,
                  b3_ref, out_ref):
    @pl.when(pl.program_id(0) == 0)
    def _():
        out_ref[...] = jnp.zeros_like(out_ref)

    h = hp_ref[0] + hp_ref[1]
    r = jnp.maximum(
        jnp.dot(h, w1_ref[...], preferred_element_type=jnp.float32) + b1_ref[...], 0.0)
    r = jnp.maximum(
        jnp.dot(r, w2_ref[...], preferred_element_type=jnp.float32) + b2_ref[...], 0.0)
    r = jnp.maximum(
        jnp.dot(r, w3_ref[...], preferred_element_type=jnp.float32) + b3_ref[...], 0.0)
    mol = mol_ref[0, 0, :]
    rows = lax.broadcasted_iota(jnp.int32, (MOLS, mol.shape[0]), 0)
    onehot = (rows == mol[None, :]).astype(jnp.float32)
    out_ref[...] += jnp.dot(onehot, r, preferred_element_type=jnp.float32)


def _readout(hparts, mol3, w1, b1, w2, b2, w3, b3, bn=1000):
    P, n, d = hparts.shape
    h = w1.shape[1]
    o = w3.shape[1]
    g = n // bn
    return pl.pallas_call(
        _readout_body,
        grid=(g,),
        in_specs=[
            pl.BlockSpec((P, bn, d), lambda i: (0, i, 0)),
            pl.BlockSpec((1, 1, bn), lambda i: (i, 0, 0)),
            pl.BlockSpec((d, h), lambda i: (0, 0)),
            pl.BlockSpec((1, h), lambda i: (0, 0)),
            pl.BlockSpec((h, h), lambda i: (0, 0)),
            pl.BlockSpec((1, h), lambda i: (0, 0)),
            pl.BlockSpec((h, o), lambda i: (0, 0)),
            pl.BlockSpec((1, o), lambda i: (0, 0)),
        ],
        out_specs=pl.BlockSpec((MOLS, o), lambda i: (0, 0)),
        out_shape=jax.ShapeDtypeStruct((MOLS, o), jnp.float32),
    )(hparts, mol3, w1, b1, w2, b2, w3, b3)


def kernel(x, edge_index, mol_ids, msg_W1, msg_b1, msg_W2, msg_b2, msg_W3,
           msg_b3, fc1_W, fc1_b, fc2_W, fc2_b, out_W, out_b):
    n, d = x.shape
    e = edge_index.shape[1]
    steps, hid, _ = msg_W1.shape

    src = edge_index[0]
    dst = edge_index[1]
    pad = EP - e
    padz = jnp.zeros((pad,), jnp.int32)
    ia2 = jnp.concatenate([dst * 2, padz]).reshape(EP // CG, CG)
    ib2 = jnp.concatenate([src * 2 + 1, padz]).reshape(EP // CG, CG)
    dstp = jnp.concatenate([dst, jnp.full((pad,), n, jnp.int32)])
    dste2 = dstp[0::2].reshape(EP // 2 // CS, CS)
    dsto2 = dstp[1::2].reshape(EP // 2 // CS, CS)
    zeros = jnp.zeros((n + 8, d), jnp.float32)

    hparts = x[None]
    for s in range(steps):
        w1 = msg_W1[s]
        wcat = jnp.concatenate([w1[:, :d].T, w1[:, d:].T], axis=1)
        ab = _node_proj(hparts, wcat)
        table = ab.reshape(2 * n, hid)
        pre_a, pre_b = _sc_gather(table, ia2, ib2)
        pap = pre_a.reshape(EP // 2, 2 * hid)
        pbp = pre_b.reshape(EP // 2, 2 * hid)
        w2t = msg_W2[s].T
        w2bd = jnp.zeros((2 * hid, 2 * hid), jnp.float32)
        w2bd = w2bd.at[:hid, :hid].set(w2t).at[hid:, hid:].set(w2t)
        b1c = jnp.tile(msg_b1[s], 2)[None, :]
        b2c = jnp.tile(msg_b2[s], 2)[None, :]
        m3a, m3b = _edge_mlp(pap, pbp, b1c, w2bd, b2c, msg_W3[s].T,
                             msg_b3[s][None, :])
        hparts = _sc_scatter(m3a, m3b, dste2, dsto2, zeros)

    mol3 = mol_ids.reshape(10, 1, n // 10)
    return _readout(hparts, mol3, fc1_W.T, fc1_b[None, :], fc2_W.T,
                    fc2_b[None, :], out_W.T, out_b[None, :])


# speedup vs baseline: 2.5974x; 1.0147x over previous
import functools

import jax
import jax.numpy as jnp
from jax import lax
from jax.experimental import pallas as pl
from jax.experimental.pallas import tpu as pltpu
from jax.experimental.pallas import tpu_sc as plsc

NC = 2
NS = 16
NW = NC * NS
MOLS = 256
EP = 163840
CG = 64
CWG = EP // NW // CG
CS = 64
CWS = EP // NW // CS
GRP = 4


def _proj_body(P, hp_ref, w_ref, out_ref):
    h = hp_ref[0]
    for p in range(1, P):
        h = h + hp_ref[p]
    out_ref[...] = jnp.dot(h, w_ref[...], preferred_element_type=jnp.float32)


def _node_proj(hparts, wcat, bn=2000):
    P, n, d = hparts.shape
    d2 = wcat.shape[1]
    return pl.pallas_call(
        functools.partial(_proj_body, P),
        grid=(n // bn,),
        in_specs=[
            pl.BlockSpec((P, bn, d), lambda i: (0, i, 0)),
            pl.BlockSpec((d, d2), lambda i: (0, 0)),
        ],
        out_specs=pl.BlockSpec((bn, d2), lambda i: (i, 0)),
        out_shape=jax.ShapeDtypeStruct((n, d2), jnp.float32),
    )(hparts, wcat)


def _sc_gather(table, ia2, ib2):
    n2, h = table.shape
    trows = (n2 // NS) // 8 * 8
    ttail = n2 - NS * trows
    mesh = plsc.VectorSubcoreMesh(core_axis_name="c", subcore_axis_name="s")

    @functools.partial(
        pl.kernel,
        out_type=(jax.ShapeDtypeStruct((EP, h), jnp.float32),
                  jax.ShapeDtypeStruct((EP, h), jnp.float32)),
        mesh=mesh,
        compiler_params=pltpu.CompilerParams(use_tc_tiling_on_sc=False),
        scratch_types=[
            pltpu.VMEM((CWG, CG), jnp.int32),
            pltpu.VMEM((CWG, CG), jnp.int32),
            pltpu.VMEM((GRP, CG, h), jnp.float32),
            pltpu.VMEM((GRP, CG, h), jnp.float32),
            pltpu.VMEM_SHARED((n2, h), jnp.float32),
            [pltpu.SemaphoreType.DMA] * GRP,
            [pltpu.SemaphoreType.DMA] * GRP,
            [pltpu.SemaphoreType.DMA] * GRP,
            [pltpu.SemaphoreType.DMA] * GRP,
        ],
    )
    def k(t_ref, ia_ref, ib_ref, oa_ref, ob_ref, iav, ibv, abuf, bbuf, tsh,
          ga, gb, sa, sb):
        ci = lax.axis_index("c")
        s = lax.axis_index("s")
        wid = s * NC + ci
        pltpu.sync_copy(t_ref.at[pl.ds(s * trows, trows)],
                        tsh.at[pl.ds(s * trows, trows)])
        if ttail:
            @pl.when(s == NS - 1)
            def _():
                pltpu.sync_copy(t_ref.at[pl.ds(NS * trows, ttail)],
                                tsh.at[pl.ds(NS * trows, ttail)])
        pltpu.sync_copy(ia_ref.at[pl.ds(wid * CWG, CWG)], iav)
        pltpu.sync_copy(ib_ref.at[pl.ds(wid * CWG, CWG)], ibv)
        plsc.subcore_barrier()
        base = wid * CWG

        def group(oj, carry):
            @pl.when(oj > 0)
            def _():
                for b in range(GRP):
                    pltpu.make_async_copy(
                        abuf.at[b], oa_ref.at[pl.ds(base * CG, CG)],
                        sa[b]).wait()
                    pltpu.make_async_copy(
                        bbuf.at[b], ob_ref.at[pl.ds(base * CG, CG)],
                        sb[b]).wait()
            descs = []
            for b in range(GRP):
                kk = GRP * oj + b
                descs.append(
                    (pltpu.async_copy(tsh.at[iav.at[kk]], abuf.at[b], ga[b]),
                     pltpu.async_copy(tsh.at[ibv.at[kk]], bbuf.at[b], gb[b])))
            for b in range(GRP):
                kk = GRP * oj + b
                da, db = descs[b]
                da.wait()
                db.wait()
                pltpu.async_copy(
                    abuf.at[b], oa_ref.at[pl.ds((base + kk) * CG, CG)], sa[b])
                pltpu.async_copy(
                    bbuf.at[b], ob_ref.at[pl.ds((base + kk) * CG, CG)], sb[b])
            return carry

        lax.fori_loop(0, CWG // GRP, group, 0)
        for b in range(GRP):
            pltpu.make_async_copy(
                abuf.at[b], oa_ref.at[pl.ds(base * CG, CG)], sa[b]).wait()
            pltpu.make_async_copy(
                bbuf.at[b], ob_ref.at[pl.ds(base * CG, CG)], sb[b]).wait()

    return k(table, ia2, ib2)


def _mlp_body(h, pa_ref, pb_ref, b1_ref, w2_ref, b2_ref, w3_ref, b3_ref,
              oa_ref, ob_ref):
    z = jnp.maximum(pa_ref[...] + pb_ref[...] + b1_ref[...], 0.0)
    m2 = jnp.maximum(
        jnp.dot(z, w2_ref[...], preferred_element_type=jnp.float32) + b2_ref[...], 0.0)
    me = m2[:, :h]
    mo = m2[:, h:]
    oa_ref[...] = jnp.maximum(
        jnp.dot(me, w3_ref[...], preferred_element_type=jnp.float32) + b3_ref[...], 0.0)
    ob_ref[...] = jnp.maximum(
        jnp.dot(mo, w3_ref[...], preferred_element_type=jnp.float32) + b3_ref[...], 0.0)


def _edge_mlp(pre_a, pre_b, b1c, w2bd, b2c, w3, b3, be=2048):
    ep2, h2 = pre_a.shape
    h = h2 // 2
    d = w3.shape[1]
    sds = jax.ShapeDtypeStruct((ep2, d), jnp.float32)
    return pl.pallas_call(
        functools.partial(_mlp_body, h),
        grid=(ep2 // be,),
        in_specs=[
            pl.BlockSpec((be, h2), lambda i: (i, 0)),
            pl.BlockSpec((be, h2), lambda i: (i, 0)),
            pl.BlockSpec((1, h2), lambda i: (0, 0)),
            pl.BlockSpec((h2, h2), lambda i: (0, 0)),
            pl.BlockSpec((1, h2), lambda i: (0, 0)),
            pl.BlockSpec((h, d), lambda i: (0, 0)),
            pl.BlockSpec((1, d), lambda i: (0, 0)),
        ],
        out_specs=[pl.BlockSpec((be, d), lambda i: (i, 0)),
                   pl.BlockSpec((be, d), lambda i: (i, 0))],
        out_shape=[sds, sds],
    )(pre_a, pre_b, b1c, w2bd, b2c, w3, b3)


def _sc_scatter(m3a, m3b, dste2, dsto2, zeros):
    d = m3a.shape[1]
    na = zeros.shape[0]
    n = na - 8
    cpw = m3a.shape[0] // NW // CS
    rw = (na // NS) // 8 * 8
    ztail = na - NS * rw
    wtail = n - NS * rw
    mesh = plsc.VectorSubcoreMesh(core_axis_name="c", subcore_axis_name="s")

    @functools.partial(
        pl.kernel,
        out_type=jax.ShapeDtypeStruct((NC, n, d), jnp.float32),
        mesh=mesh,
        scratch_types=[
            pltpu.VMEM((cpw, CS), jnp.int32),
            pltpu.VMEM((cpw, CS), jnp.int32),
            pltpu.VMEM((GRP, CS, d), jnp.float32),
            pltpu.VMEM_SHARED((na, d), jnp.float32),
            [pltpu.SemaphoreType.DMA] * GRP,
            [pltpu.SemaphoreType.DMA] * GRP,
        ],
    )
    def k(ma_ref, mb_ref, de_ref, do_ref, z_ref, out_ref, idxe, idxo, rbuf,
          acc, g, c):
        ci = lax.axis_index("c")
        s = lax.axis_index("s")
        wid = s * NC + ci
        pltpu.sync_copy(z_ref.at[pl.ds(s * rw, rw)], acc.at[pl.ds(s * rw, rw)])
        if ztail:
            @pl.when(s == NS - 1)
            def _():
                pltpu.sync_copy(z_ref.at[pl.ds(NS * rw, ztail)],
                                acc.at[pl.ds(NS * rw, ztail)])
        pltpu.sync_copy(de_ref.at[pl.ds(wid * cpw, cpw)], idxe)
        pltpu.sync_copy(do_ref.at[pl.ds(wid * cpw, cpw)], idxo)
        plsc.subcore_barrier()
        base = wid * cpw

        for m_ref, idxv in ((ma_ref, idxe), (mb_ref, idxo)):
            def group(oj, carry):
                descs = []
                for b in range(GRP):
                    kk = GRP * oj + b
                    descs.append(pltpu.async_copy(
                        m_ref.at[pl.ds((base + kk) * CS, CS)], rbuf.at[b],
                        g[b]))
                sdescs = []
                for b in range(GRP):
                    kk = GRP * oj + b
                    descs[b].wait()
                    sdescs.append(pltpu.async_copy(
                        rbuf.at[b], acc.at[idxv.at[kk]], c[b], add=True))
                for sd in sdescs:
                    sd.wait()
                return carry

            lax.fori_loop(0, cpw // GRP, group, 0)
        plsc.subcore_barrier()
        pltpu.sync_copy(acc.at[pl.ds(s * rw, rw)], out_ref.at[ci, pl.ds(s * rw, rw)])
        if wtail:
            @pl.when(s == NS - 1)
            def _():
                pltpu.sync_copy(acc.at[pl.ds(NS * rw, wtail)],
                                out_ref.at[ci, pl.ds(NS * rw, wtail)])

    return k(m3a, m3b, dste2, dsto2, zeros)


def _readout_body(hp_ref, mol_ref, w1_ref, b1_ref, w2_ref, b2_ref, w3_ref,
                  b3_ref, out_ref):
    @pl.when(pl.program_id(0) == 0)
    def _():
        out_ref[...] = jnp.zeros_like(out_ref)

    h = hp_ref[0] + hp_ref[1]
    r = jnp.maximum(
        jnp.dot(h, w1_ref[...], preferred_element_type=jnp.float32) + b1_ref[...], 0.0)
    r = jnp.maximum(
        jnp.dot(r, w2_ref[...], preferred_element_type=jnp.float32) + b2_ref[...], 0.0)
    r = jnp.maximum(
        jnp.dot(r, w3_ref[...], preferred_element_type=jnp.float32) + b3_ref[...], 0.0)
    mol = mol_ref[0, 0, :]
    rows = lax.broadcasted_iota(jnp.int32, (MOLS, mol.shape[0]), 0)
    onehot = (rows == mol[None, :]).astype(jnp.float32)
    out_ref[...] += jnp.dot(onehot, r, preferred_element_type=jnp.float32)


def _readout(hparts, mol3, w1, b1, w2, b2, w3, b3, bn=1000):
    P, n, d = hparts.shape
    h = w1.shape[1]
    o = w3.shape[1]
    g = n // bn
    return pl.pallas_call(
        _readout_body,
        grid=(g,),
        in_specs=[
            pl.BlockSpec((P, bn, d), lambda i: (0, i, 0)),
            pl.BlockSpec((1, 1, bn), lambda i: (i, 0, 0)),
            pl.BlockSpec((d, h), lambda i: (0, 0)),
            pl.BlockSpec((1, h), lambda i: (0, 0)),
            pl.BlockSpec((h, h), lambda i: (0, 0)),
            pl.BlockSpec((1, h), lambda i: (0, 0)),
            pl.BlockSpec((h, o), lambda i: (0, 0)),
            pl.BlockSpec((1, o), lambda i: (0, 0)),
        ],
        out_specs=pl.BlockSpec((MOLS, o), lambda i: (0, 0)),
        out_shape=jax.ShapeDtypeStruct((MOLS, o), jnp.float32),
    )(hparts, mol3, w1, b1, w2, b2, w3, b3)


def kernel(x, edge_index, mol_ids, msg_W1, msg_b1, msg_W2, msg_b2, msg_W3,
           msg_b3, fc1_W, fc1_b, fc2_W, fc2_b, out_W, out_b):
    n, d = x.shape
    e = edge_index.shape[1]
    steps, hid, _ = msg_W1.shape

    src = edge_index[0]
    dst = edge_index[1]
    pad = EP - e
    padz = jnp.zeros((pad,), jnp.int32)
    ia2 = jnp.concatenate([dst * 2, padz]).reshape(EP // CG, CG)
    ib2 = jnp.concatenate([src * 2 + 1, padz]).reshape(EP // CG, CG)
    dstp = jnp.concatenate([dst, jnp.full((pad,), n, jnp.int32)])
    dste2 = dstp[0::2].reshape(EP // 2 // CS, CS)
    dsto2 = dstp[1::2].reshape(EP // 2 // CS, CS)
    zeros = jnp.zeros((n + 8, d), jnp.float32)

    hparts = x[None]
    for s in range(steps):
        w1 = msg_W1[s]
        wcat = jnp.concatenate([w1[:, :d].T, w1[:, d:].T], axis=1)
        ab = _node_proj(hparts, wcat)
        table = ab.reshape(2 * n, hid)
        pre_a, pre_b = _sc_gather(table, ia2, ib2)
        pap = pre_a.reshape(EP // 2, 2 * hid)
        pbp = pre_b.reshape(EP // 2, 2 * hid)
        w2t = msg_W2[s].T
        w2bd = jnp.zeros((2 * hid, 2 * hid), jnp.float32)
        w2bd = w2bd.at[:hid, :hid].set(w2t).at[hid:, hid:].set(w2t)
        b1c = jnp.tile(msg_b1[s], 2)[None, :]
        b2c = jnp.tile(msg_b2[s], 2)[None, :]
        m3a, m3b = _edge_mlp(pap, pbp, b1c, w2bd, b2c, msg_W3[s].T,
                             msg_b3[s][None, :])
        hparts = _sc_scatter(m3a, m3b, dste2, dsto2, zeros)

    mol3 = mol_ids.reshape(10, 1, n // 10)
    return _readout(hparts, mol3, fc1_W.T, fc1_b[None, :], fc2_W.T,
                    fc2_b[None, :], out_W.T, out_b[None, :])


# speedup vs baseline: 2.7599x; 1.0625x over previous
import functools

import jax
import jax.numpy as jnp
from jax import lax
from jax.experimental import pallas as pl
from jax.experimental.pallas import tpu as pltpu
from jax.experimental.pallas import tpu_sc as plsc

NC = 2
NS = 16
NW = NC * NS
MOLS = 256
EP = 163840
CG = 64
CWG = EP // NW // CG
CS = 64
CWS = EP // NW // CS
GRP = 4


def _proj_body(P, hp_ref, w_ref, out_ref):
    h = hp_ref[0]
    for p in range(1, P):
        h = h + hp_ref[p]
    out_ref[...] = jnp.dot(h, w_ref[...], preferred_element_type=jnp.float32)


def _node_proj(hparts, wcat, bn=2000):
    P, n, d = hparts.shape
    d2 = wcat.shape[1]
    return pl.pallas_call(
        functools.partial(_proj_body, P),
        grid=(n // bn,),
        in_specs=[
            pl.BlockSpec((P, bn, d), lambda i: (0, i, 0)),
            pl.BlockSpec((d, d2), lambda i: (0, 0)),
        ],
        out_specs=pl.BlockSpec((bn, d2), lambda i: (i, 0)),
        out_shape=jax.ShapeDtypeStruct((n, d2), jnp.float32),
    )(hparts, wcat)


def _sc_gather(table, ia2, ib2):
    n2, h = table.shape
    trows = (n2 // NS) // 8 * 8
    ttail = n2 - NS * trows
    mesh = plsc.VectorSubcoreMesh(core_axis_name="c", subcore_axis_name="s")

    @functools.partial(
        pl.kernel,
        out_type=(jax.ShapeDtypeStruct((EP, h), jnp.float32),
                  jax.ShapeDtypeStruct((EP, h), jnp.float32)),
        mesh=mesh,
        compiler_params=pltpu.CompilerParams(use_tc_tiling_on_sc=False),
        scratch_types=[
            pltpu.VMEM((CWG, CG), jnp.int32),
            pltpu.VMEM((CWG, CG), jnp.int32),
            pltpu.VMEM((GRP, CG, h), jnp.float32),
            pltpu.VMEM((GRP, CG, h), jnp.float32),
            pltpu.VMEM_SHARED((n2, h), jnp.float32),
            [pltpu.SemaphoreType.DMA] * GRP,
            [pltpu.SemaphoreType.DMA] * GRP,
            [pltpu.SemaphoreType.DMA] * GRP,
            [pltpu.SemaphoreType.DMA] * GRP,
        ],
    )
    def k(t_ref, ia_ref, ib_ref, oa_ref, ob_ref, iav, ibv, abuf, bbuf, tsh,
          ga, gb, sa, sb):
        ci = lax.axis_index("c")
        s = lax.axis_index("s")
        wid = s * NC + ci
        pltpu.sync_copy(t_ref.at[pl.ds(s * trows, trows)],
                        tsh.at[pl.ds(s * trows, trows)])
        if ttail:
            @pl.when(s == NS - 1)
            def _():
                pltpu.sync_copy(t_ref.at[pl.ds(NS * trows, ttail)],
                                tsh.at[pl.ds(NS * trows, ttail)])
        pltpu.sync_copy(ia_ref.at[pl.ds(wid * CWG, CWG)], iav)
        pltpu.sync_copy(ib_ref.at[pl.ds(wid * CWG, CWG)], ibv)
        plsc.subcore_barrier()
        base = wid * CWG

        def group(oj, carry):
            @pl.when(oj > 0)
            def _():
                for b in range(GRP):
                    pltpu.make_async_copy(
                        abuf.at[b], oa_ref.at[pl.ds(base * CG, CG)],
                        sa[b]).wait()
                    pltpu.make_async_copy(
                        bbuf.at[b], ob_ref.at[pl.ds(base * CG, CG)],
                        sb[b]).wait()
            descs = []
            for b in range(GRP):
                kk = GRP * oj + b
                descs.append(
                    (pltpu.async_copy(tsh.at[iav.at[kk]], abuf.at[b], ga[b]),
                     pltpu.async_copy(tsh.at[ibv.at[kk]], bbuf.at[b], gb[b])))
            for b in range(GRP):
                kk = GRP * oj + b
                da, db = descs[b]
                da.wait()
                db.wait()
                pltpu.async_copy(
                    abuf.at[b], oa_ref.at[pl.ds((base + kk) * CG, CG)], sa[b])
                pltpu.async_copy(
                    bbuf.at[b], ob_ref.at[pl.ds((base + kk) * CG, CG)], sb[b])
            return carry

        lax.fori_loop(0, CWG // GRP, group, 0)
        for b in range(GRP):
            pltpu.make_async_copy(
                abuf.at[b], oa_ref.at[pl.ds(base * CG, CG)], sa[b]).wait()
            pltpu.make_async_copy(
                bbuf.at[b], ob_ref.at[pl.ds(base * CG, CG)], sb[b]).wait()

    return k(table, ia2, ib2)


def _mlp_body(h, pa_ref, pb_ref, b1_ref, w2_ref, b2_ref, w3_ref, b3_ref,
              oa_ref, ob_ref):
    z = jnp.maximum(pa_ref[...] + pb_ref[...] + b1_ref[...], 0.0)
    m2 = jnp.maximum(
        jnp.dot(z, w2_ref[...], preferred_element_type=jnp.float32) + b2_ref[...], 0.0)
    me = m2[:, :h]
    mo = m2[:, h:]
    oa_ref[...] = jnp.maximum(
        jnp.dot(me, w3_ref[...], preferred_element_type=jnp.float32) + b3_ref[...], 0.0)
    ob_ref[...] = jnp.maximum(
        jnp.dot(mo, w3_ref[...], preferred_element_type=jnp.float32) + b3_ref[...], 0.0)


def _edge_mlp(pre_a, pre_b, b1c, w2bd, b2c, w3, b3, be=4096):
    ep2, h2 = pre_a.shape
    h = h2 // 2
    d = w3.shape[1]
    sds = jax.ShapeDtypeStruct((ep2, d), jnp.float32)
    return pl.pallas_call(
        functools.partial(_mlp_body, h),
        grid=(ep2 // be,),
        in_specs=[
            pl.BlockSpec((be, h2), lambda i: (i, 0)),
            pl.BlockSpec((be, h2), lambda i: (i, 0)),
            pl.BlockSpec((1, h2), lambda i: (0, 0)),
            pl.BlockSpec((h2, h2), lambda i: (0, 0)),
            pl.BlockSpec((1, h2), lambda i: (0, 0)),
            pl.BlockSpec((h, d), lambda i: (0, 0)),
            pl.BlockSpec((1, d), lambda i: (0, 0)),
        ],
        out_specs=[pl.BlockSpec((be, d), lambda i: (i, 0)),
                   pl.BlockSpec((be, d), lambda i: (i, 0))],
        out_shape=[sds, sds],
    )(pre_a, pre_b, b1c, w2bd, b2c, w3, b3)


def _sc_scatter(m3a, m3b, dste2, dsto2, zeros):
    d = m3a.shape[1]
    na = zeros.shape[0]
    n = na - 8
    cpw = m3a.shape[0] // NW // CS
    rw = (na // NS) // 8 * 8
    ztail = na - NS * rw
    wtail = n - NS * rw
    mesh = plsc.VectorSubcoreMesh(core_axis_name="c", subcore_axis_name="s")

    @functools.partial(
        pl.kernel,
        out_type=jax.ShapeDtypeStruct((NC, n, d), jnp.float32),
        mesh=mesh,
        scratch_types=[
            pltpu.VMEM((cpw, CS), jnp.int32),
            pltpu.VMEM((cpw, CS), jnp.int32),
            pltpu.VMEM((GRP, CS, d), jnp.float32),
            pltpu.VMEM_SHARED((na, d), jnp.float32),
            [pltpu.SemaphoreType.DMA] * GRP,
            [pltpu.SemaphoreType.DMA] * GRP,
            pltpu.SemaphoreType.DMA,
        ],
    )
    def k(ma_ref, mb_ref, de_ref, do_ref, z_ref, out_ref, idxe, idxo, rbuf,
          acc, g, c, zs):
        ci = lax.axis_index("c")
        s = lax.axis_index("s")
        wid = s * NC + ci
        zd = pltpu.async_copy(z_ref.at[pl.ds(s * rw, rw)],
                              acc.at[pl.ds(s * rw, rw)], zs)
        pltpu.sync_copy(de_ref.at[pl.ds(wid * cpw, cpw)], idxe)
        pltpu.sync_copy(do_ref.at[pl.ds(wid * cpw, cpw)], idxo)
        zd.wait()
        if ztail:
            @pl.when(s == NS - 1)
            def _():
                pltpu.sync_copy(z_ref.at[pl.ds(NS * rw, ztail)],
                                acc.at[pl.ds(NS * rw, ztail)])
        plsc.subcore_barrier()
        base = wid * cpw

        for m_ref, idxv in ((ma_ref, idxe), (mb_ref, idxo)):
            def group(oj, carry):
                descs = []
                for b in range(GRP):
                    kk = GRP * oj + b
                    descs.append(pltpu.async_copy(
                        m_ref.at[pl.ds((base + kk) * CS, CS)], rbuf.at[b],
                        g[b]))
                sdescs = []
                for b in range(GRP):
                    kk = GRP * oj + b
                    descs[b].wait()
                    sdescs.append(pltpu.async_copy(
                        rbuf.at[b], acc.at[idxv.at[kk]], c[b], add=True))
                for sd in sdescs:
                    sd.wait()
                return carry

            lax.fori_loop(0, cpw // GRP, group, 0)
        plsc.subcore_barrier()
        pltpu.sync_copy(acc.at[pl.ds(s * rw, rw)], out_ref.at[ci, pl.ds(s * rw, rw)])
        if wtail:
            @pl.when(s == NS - 1)
            def _():
                pltpu.sync_copy(acc.at[pl.ds(NS * rw, wtail)],
                                out_ref.at[ci, pl.ds(NS * rw, wtail)])

    return k(m3a, m3b, dste2, dsto2, zeros)


def _readout_body(hp_ref, mol_ref, w1_ref, b1_ref, w2_ref, b2_ref, w3_ref,
                  b3_ref, out_ref):
    @pl.when(pl.program_id(0) == 0)
    def _():
        out_ref[...] = jnp.zeros_like(out_ref)

    h = hp_ref[0] + hp_ref[1]
    r = jnp.maximum(
        jnp.dot(h, w1_ref[...], preferred_element_type=jnp.float32) + b1_ref[...], 0.0)
    r = jnp.maximum(
        jnp.dot(r, w2_ref[...], preferred_element_type=jnp.float32) + b2_ref[...], 0.0)
    r = jnp.maximum(
        jnp.dot(r, w3_ref[...], preferred_element_type=jnp.float32) + b3_ref[...], 0.0)
    mol = mol_ref[0, 0, :]
    rows = lax.broadcasted_iota(jnp.int32, (MOLS, mol.shape[0]), 0)
    onehot = (rows == mol[None, :]).astype(jnp.float32)
    out_ref[...] += jnp.dot(onehot, r, preferred_element_type=jnp.float32)


def _readout(hparts, mol3, w1, b1, w2, b2, w3, b3, bn=1000):
    P, n, d = hparts.shape
    h = w1.shape[1]
    o = w3.shape[1]
    g = n // bn
    return pl.pallas_call(
        _readout_body,
        grid=(g,),
        in_specs=[
            pl.BlockSpec((P, bn, d), lambda i: (0, i, 0)),
            pl.BlockSpec((1, 1, bn), lambda i: (i, 0, 0)),
            pl.BlockSpec((d, h), lambda i: (0, 0)),
            pl.BlockSpec((1, h), lambda i: (0, 0)),
            pl.BlockSpec((h, h), lambda i: (0, 0)),
            pl.BlockSpec((1, h), lambda i: (0, 0)),
            pl.BlockSpec((h, o), lambda i: (0, 0)),
            pl.BlockSpec((1, o), lambda i: (0, 0)),
        ],
        out_specs=pl.BlockSpec((MOLS, o), lambda i: (0, 0)),
        out_shape=jax.ShapeDtypeStruct((MOLS, o), jnp.float32),
    )(hparts, mol3, w1, b1, w2, b2, w3, b3)


def kernel(x, edge_index, mol_ids, msg_W1, msg_b1, msg_W2, msg_b2, msg_W3,
           msg_b3, fc1_W, fc1_b, fc2_W, fc2_b, out_W, out_b):
    n, d = x.shape
    e = edge_index.shape[1]
    steps, hid, _ = msg_W1.shape

    src = edge_index[0]
    dst = edge_index[1]
    pad = EP - e
    padz = jnp.zeros((pad,), jnp.int32)
    ia2 = jnp.concatenate([dst * 2, padz]).reshape(EP // CG, CG)
    ib2 = jnp.concatenate([src * 2 + 1, padz]).reshape(EP // CG, CG)
    dstp = jnp.concatenate([dst, jnp.full((pad,), n, jnp.int32)])
    dste2 = dstp[0::2].reshape(EP // 2 // CS, CS)
    dsto2 = dstp[1::2].reshape(EP // 2 // CS, CS)
    zeros = jnp.zeros((n + 8, d), jnp.float32)

    hparts = x[None]
    for s in range(steps):
        w1 = msg_W1[s]
        wcat = jnp.concatenate([w1[:, :d].T, w1[:, d:].T], axis=1)
        ab = _node_proj(hparts, wcat)
        table = ab.reshape(2 * n, hid)
        pre_a, pre_b = _sc_gather(table, ia2, ib2)
        pap = pre_a.reshape(EP // 2, 2 * hid)
        pbp = pre_b.reshape(EP // 2, 2 * hid)
        w2t = msg_W2[s].T
        w2bd = jnp.zeros((2 * hid, 2 * hid), jnp.float32)
        w2bd = w2bd.at[:hid, :hid].set(w2t).at[hid:, hid:].set(w2t)
        b1c = jnp.tile(msg_b1[s], 2)[None, :]
        b2c = jnp.tile(msg_b2[s], 2)[None, :]
        m3a, m3b = _edge_mlp(pap, pbp, b1c, w2bd, b2c, msg_W3[s].T,
                             msg_b3[s][None, :])
        hparts = _sc_scatter(m3a, m3b, dste2, dsto2, zeros)

    mol3 = mol_ids.reshape(10, 1, n // 10)
    return _readout(hparts, mol3, fc1_W.T, fc1_b[None, :], fc2_W.T,
                    fc2_b[None, :], out_W.T, out_b[None, :])
